# Initial kernel scaffold; baseline (speedup 1.0000x reference)
import jax
import jax.numpy as jnp
from jax.experimental import pallas as pl


def kernel(x, edge_index, edge_attr, batch, demographics, emb, l0_Wl, l0_Wr, l0_We, l0_att, l0_b, l0_Wres, l1_Wl, l1_Wr, l1_We, l1_att, l1_b, l1_Wres, l2_Wl, l2_Wr, l2_We, l2_att, l2_b, l2_Wres, gn0_w, gn0_b, gn0_ms, gn1_w, gn1_b, gn1_ms, demo_W, demo_b, c1_W, c1_b, bn1_w, bn1_b, c2_W, c2_b, bn2_w, bn2_b, c3_W, c3_b):
    raise NotImplementedError("write your pallas kernel here")


# speedup vs baseline: 1.1632x; 1.1632x over previous
import functools

import jax
import jax.numpy as jnp
from jax.experimental import pallas as pl

N = 50000
E = 800000
G = 16
H = 4
C = 16
HC = 64


def _lrelu(v, a):
    return jnp.where(v >= 0, v, a * v)


def _gat_layer(h, src, dst, eattr, mean_attr, Wl, Wr, We, att, b, Wres):
    xl = (h @ Wl).reshape(N, H, C)
    xr = (h @ Wr).reshape(N, H, C)
    ee_loop = (mean_attr @ We).reshape(1, H, C)
    sl_logit = jnp.sum(_lrelu(xl + xr + ee_loop, 0.2) * att[None], axis=-1)
    ee = (eattr @ We).reshape(E, H, C)
    m = _lrelu(xl[src] + xr[dst] + ee, 0.2)
    logits = jnp.sum(m * att[None], axis=-1)
    ex = jnp.exp(logits - sl_logit[dst])
    den = jax.ops.segment_sum(ex, dst, num_segments=N) + 1.0
    num = jax.ops.segment_sum(ex[:, :, None] * xl[src], dst, num_segments=N) + xl
    out = num / (den[:, :, None] + 1e-16)
    return out.reshape(N, HC) + h @ Wres + b


def _graph_norm(h, onehot, cnt, w, b, ms):
    mu = (onehot.T @ h) / cnt[:, None]
    sub = h - ms[None, :] * (onehot @ mu)
    var = (onehot.T @ (sub * sub)) / cnt[:, None]
    return w * sub / jnp.sqrt(onehot @ var + 1e-5) + b


def _head_kernel(z_ref, c1w, c1b, s1w, s1b, c2w, c2b, s2w, s2b, c3w, c3b, o_ref):
    z = z_ref[...]
    z = z @ c1w[...] + c1b[...]
    z = z * s1w[...] + s1b[...]
    z = _lrelu(z, 0.01)
    z = z @ c2w[...] + c2b[...]
    z = z * s2w[...] + s2b[...]
    z = _lrelu(z, 0.01)
    o_ref[...] = z @ c3w[...] + c3b[...]


def kernel(x, edge_index, edge_attr, batch, demographics, emb,
           l0_Wl, l0_Wr, l0_We, l0_att, l0_b, l0_Wres,
           l1_Wl, l1_Wr, l1_We, l1_att, l1_b, l1_Wres,
           l2_Wl, l2_Wr, l2_We, l2_att, l2_b, l2_Wres,
           gn0_w, gn0_b, gn0_ms, gn1_w, gn1_b, gn1_ms,
           demo_W, demo_b, c1_W, c1_b, bn1_w, bn1_b,
           c2_W, c2_b, bn2_w, bn2_b, c3_W, c3_b):
    src = edge_index[0]
    dst = edge_index[1]
    mean_attr = jnp.mean(edge_attr, axis=0, keepdims=True)
    h = emb[x]
    onehot = (batch[:, None] == jnp.arange(G)[None, :]).astype(jnp.float32)
    cnt = jnp.maximum(jnp.sum(onehot, axis=0), 1.0)

    layers = [
        (l0_Wl, l0_Wr, l0_We, l0_att, l0_b, l0_Wres),
        (l1_Wl, l1_Wr, l1_We, l1_att, l1_b, l1_Wres),
        (l2_Wl, l2_Wr, l2_We, l2_att, l2_b, l2_Wres),
    ]
    gns = [(gn0_w, gn0_b, gn0_ms), (gn1_w, gn1_b, gn1_ms)]
    for i in range(3):
        h = _gat_layer(h, src, dst, edge_attr, mean_attr, *layers[i])
        if i < 2:
            h = _graph_norm(h, onehot, cnt, *gns[i])
            h = jax.nn.elu(h)

    gm = (onehot.T @ h) / cnt[:, None]
    demo = demographics @ demo_W + demo_b
    z = jnp.concatenate([gm, demo], axis=1)
    inv1 = (1.0 / jnp.sqrt(1.0 + 1e-5)) * bn1_w
    inv2 = (1.0 / jnp.sqrt(1.0 + 1e-5)) * bn2_w
    out = pl.pallas_call(
        _head_kernel,
        out_shape=jax.ShapeDtypeStruct((G, 2), jnp.float32),
    )(z, c1_W, c1_b, inv1, bn1_b, c2_W, c2_b, inv2, bn2_b, c3_W, c3_b)
    return out


# speedup vs baseline: 27.1066x; 23.3044x over previous
import numpy as np
import jax
import jax.numpy as jnp
from jax import lax
from jax.experimental import pallas as pl
from jax.experimental.pallas import tpu as pltpu
from jax.experimental.pallas import tpu_sc as plsc

N = 50000
E = 800000
G = 16
H = 4
C = 16
HC = 64

NC = 2
NS = 16
L = 16
ROW = 40
BE = 32
NBAT = E // BE
TB = -(-NBAT // NS)
RPT = N // NS
NP = 50048
BN = 1000
NBLK = N // BN

_f32 = jnp.float32


def _col(k):
    return k + 4 * (k // 16)


def _selectors():
    Pxl = []
    Psll = []
    Pn = []
    Db = []
    for c in range(2):
        pxl = np.zeros((64, ROW), np.float32)
        psll = np.zeros((64, ROW), np.float32)
        pn = np.zeros((ROW, 64), np.float32)
        db = np.zeros((ROW, 64), np.float32)
        for k in range(32):
            pxl[32 * c + k, _col(k)] = 1.0
            pn[_col(k), 32 * c + k] = 1.0
            db[16 + 20 * (k // 16), 32 * c + k] = 1.0
        psll[32 * c, 16] = 1.0
        psll[32 * c + 16, 36] = 1.0
        Pxl.append(pxl)
        Psll.append(psll)
        Pn.append(pn)
        Db.append(db)
    ones_row = np.zeros((1, ROW), np.float32)
    ones_row[0, 16] = 1.0
    ones_row[0, 36] = 1.0
    Bmat = np.kron(np.eye(4, dtype=np.float32), np.ones((16, 16), np.float32))
    return Pxl, Psll, Pn, Db, ones_row, Bmat


_PXL, _PSLL, _PN, _DB, _ONESROW, _BMAT = _selectors()


def _sel_args():
    return (jnp.asarray(_BMAT), jnp.asarray(_PXL[0]), jnp.asarray(_PXL[1]),
            jnp.asarray(_PSLL[0]), jnp.asarray(_PSLL[1]),
            jnp.asarray(_ONESROW))


def _pn_args():
    return (jnp.asarray(_PN[0]), jnp.asarray(_PN[1]),
            jnp.asarray(_DB[0]), jnp.asarray(_DB[1]))


_SEL_SPECS = [None, None, None, None, None, None]
_PN_SPECS = [None, None, None, None]


def _emb_gather_body(emb_h, x_h, out_h, idx_v, rows_v, sem):
    c = lax.axis_index("c")
    s = lax.axis_index("s")
    wid = s * NC + c

    def body(j, carry):
        bid = wid + NC * NS * j

        @pl.when(bid < NP // BE)
        def _():
            off = bid * BE
            pltpu.sync_copy(x_h.at[pl.ds(off, BE)], idx_v)
            pltpu.async_copy(emb_h.at[idx_v], rows_v, sem).wait()
            pltpu.sync_copy(rows_v, out_h.at[pl.ds(off, BE)])

        return carry

    lax.fori_loop(0, -(-(NP // BE) // (NC * NS)), body, 0)


def _emb_gather(emb, xpad):
    mesh = plsc.VectorSubcoreMesh(core_axis_name="c", subcore_axis_name="s")
    return pl.kernel(
        _emb_gather_body,
        out_type=jax.ShapeDtypeStruct((NP, 16), _f32),
        mesh=mesh,
        compiler_params=pltpu.CompilerParams(use_tc_tiling_on_sc=False, needs_layout_passes=False),
        scratch_types=[
            pltpu.VMEM((BE,), jnp.int32),
            pltpu.VMEM((BE, 16), _f32),
            pltpu.SemaphoreType.DMA,
        ],
    )(emb, xpad)


def _edge_body(xl0, xl1, xr0, xr1, srcr, dstr, ear, constr,
               out0, out1,
               acc, xl_st, xr_st, out_st, gsrc, gdst, ea_st, cvm,
               sema, semb):
    c = lax.axis_index("c")
    s = lax.axis_index("s")
    pltpu.sync_copy(constr.at[c], cvm)

    def _blkcopy(src_ref, dst_ref):
        pltpu.sync_copy(src_ref.at[pl.ds(s * 2960, 2960)],
                        dst_ref.at[pl.ds(s * 2960, 2960)])

        @pl.when(s == 0)
        def _():
            pltpu.sync_copy(src_ref.at[pl.ds(47360, 2640)],
                            dst_ref.at[pl.ds(47360, 2640)])

    @pl.when(c == 0)
    def _():
        _blkcopy(xl0, acc)

    @pl.when(c == 1)
    def _():
        _blkcopy(xl1, acc)

    riota = lax.iota(jnp.int32, L)
    zv = jnp.zeros((L,), _f32)
    for col in (17, 18, 19, 37, 38, 39):
        colv = jnp.full((L,), col, jnp.int32)
        for g in range(BE // L):
            plsc.store_scatter(out_st, [riota + g * L, colv], zv)

    plsc.subcore_barrier()

    wevs = [cvm[pl.ds(0, 16)], cvm[pl.ds(16, 16)]]
    atvs = [cvm[pl.ds(32, 16)], cvm[pl.ds(48, 16)]]

    def batch(j, carry):
        bid = s + NS * j

        @pl.when(bid < NBAT)
        def _():
            off = bid * BE
            pltpu.sync_copy(srcr.at[pl.ds(off, BE)], gsrc)
            pltpu.sync_copy(dstr.at[pl.ds(off, BE)], gdst)
            pltpu.sync_copy(ear.at[pl.ds(off, BE)], ea_st)

            @pl.when(c == 0)
            def _():
                ca = pltpu.async_copy(xl0.at[gsrc], xl_st, sema)
                cb = pltpu.async_copy(xr0.at[gdst], xr_st, semb)
                ca.wait()
                cb.wait()

            @pl.when(c == 1)
            def _():
                ca = pltpu.async_copy(xl1.at[gsrc], xl_st, sema)
                cb = pltpu.async_copy(xr1.at[gdst], xr_st, semb)
                ca.wait()
                cb.wait()

            for g in range(BE // L):
                rows = riota + g * L
                eav = ea_st[pl.ds(g * L, L)]
                for hh in range(2):
                    base = 20 * hh
                    logit = jnp.zeros((L,), _f32)
                    xls = []
                    for ch in range(16):
                        colv = jnp.full((L,), base + ch, jnp.int32)
                        xlc = plsc.load_gather(xl_st, [rows, colv])
                        xrc = plsc.load_gather(xr_st, [rows, colv])
                        pre = xlc + xrc + eav * wevs[hh][ch]
                        m = jnp.maximum(pre, 0.2 * pre)
                        logit = logit + m * atvs[hh][ch]
                        xls.append(xlc)
                    dencol = jnp.full((L,), base + 16, jnp.int32)
                    sll = plsc.load_gather(xr_st, [rows, dencol])
                    ex = jnp.exp(logit - sll)
                    for ch in range(16):
                        colv = jnp.full((L,), base + ch, jnp.int32)
                        plsc.store_scatter(out_st, [rows, colv], ex * xls[ch])
                    plsc.store_scatter(out_st, [rows, dencol], ex)

            pltpu.sync_copy(out_st, acc.at[gdst], add=True)

        return carry

    lax.fori_loop(0, TB, batch, 0)
    plsc.subcore_barrier()

    @pl.when(c == 0)
    def _():
        _blkcopy(acc, out0)

    @pl.when(c == 1)
    def _():
        _blkcopy(acc, out1)


def _edge_call(xl0, xl1, xr0, xr1, src, dst, ea, constT):
    mesh = plsc.VectorSubcoreMesh(core_axis_name="c", subcore_axis_name="s")
    return pl.kernel(
        _edge_body,
        out_type=[
            jax.ShapeDtypeStruct((N, ROW), _f32),
            jax.ShapeDtypeStruct((N, ROW), _f32),
        ],
        mesh=mesh,
        compiler_params=pltpu.CompilerParams(use_tc_tiling_on_sc=False, needs_layout_passes=False),
        scratch_types=[
            pltpu.VMEM_SHARED((N, ROW), _f32),
            pltpu.VMEM((BE, ROW), _f32),
            pltpu.VMEM((BE, ROW), _f32),
            pltpu.VMEM((BE, ROW), _f32),
            pltpu.VMEM((BE,), jnp.int32),
            pltpu.VMEM((BE,), jnp.int32),
            pltpu.VMEM((BE,), _f32),
            pltpu.VMEM((64,), _f32),
            pltpu.SemaphoreType.DMA,
            pltpu.SemaphoreType.DMA,
        ],
    )(xl0, xl1, xr0, xr1, src, dst, ea, constT)


def _sum_kern(ea_ref, o_ref):
    @pl.when(pl.program_id(0) == 0)
    def _():
        o_ref[...] = jnp.zeros_like(o_ref)

    o_ref[...] += jnp.sum(ea_ref[...])


def _tables_from_h(hb, Wl, Wr, We, att_flat, ea_mean, sel):
    bmat, pxl0, pxl1, psll0, psll1, onesrow = sel
    xl = hb @ Wl
    xr = hb @ Wr
    pre = xl + xr + ea_mean * We
    m = jnp.maximum(pre, 0.2 * pre)
    sll_wide = (m * att_flat) @ bmat
    xlT0 = xl @ pxl0 + onesrow
    xlT1 = xl @ pxl1 + onesrow
    xrT0 = xr @ pxl0 + sll_wide @ psll0
    xrT1 = xr @ pxl1 + sll_wide @ psll1
    return xlT0, xlT1, xrT0, xrT1


def _k1_kern(h_ref, Wl, Wr, We, att, eam,
             bmat, pxl0, pxl1, psll0, psll1, onesrow, o0, o1, o2, o3):
    sel = (bmat[...], pxl0[...], pxl1[...], psll0[...], psll1[...],
           onesrow[...])
    t = _tables_from_h(h_ref[...], Wl[...], Wr[...], We[...], att[...],
                       eam[0, 0], sel)
    o0[...], o1[...], o2[...], o3[...] = t


def _assemble_h(o0b, o1b, pn0, pn1, db0, db1):
    h = jnp.zeros((o0b.shape[0], HC), _f32)
    for ob, pn, db in ((o0b, pn0, db0), (o1b, pn1, db1)):
        h = h + (ob @ pn) / (ob @ db + 1e-16)
    return h


def _e1_kern(o0, o1, h_ref, oh_ref, Wres, bias, pn0, pn1, db0, db1,
             hn_ref, mu_ref, cnt_ref):
    hn = _assemble_h(o0[...], o1[...], pn0[...], pn1[...], db0[...],
                     db1[...]) + h_ref[...] @ Wres[...] + bias[...]
    hn_ref[...] = hn
    oh = oh_ref[...]

    @pl.when(pl.program_id(0) == 0)
    def _():
        mu_ref[...] = jnp.zeros_like(mu_ref)
        cnt_ref[...] = jnp.zeros_like(cnt_ref)

    mu_ref[...] += lax.dot_general(oh, hn, (((0,), (0,)), ((), ())))
    cnt_ref[...] += lax.dot_general(
        oh, jnp.ones_like(hn), (((0,), (0,)), ((), ())))


def _e2_kern(hn_ref, oh_ref, mus, cnt, ms, var_ref):
    cntc = jnp.maximum(cnt[...], 1.0)
    mu = mus[...] / cntc
    oh = oh_ref[...]
    sub = hn_ref[...] - ms[...] * (oh @ mu)

    @pl.when(pl.program_id(0) == 0)
    def _():
        var_ref[...] = jnp.zeros_like(var_ref)

    var_ref[...] += lax.dot_general(oh, sub * sub, (((0,), (0,)), ((), ())))


def _e3_kern(hn_ref, oh_ref, mus, cnt, vars_, gw, gb, ms,
             Wl, Wr, We, att, eam,
             bmat, pxl0, pxl1, psll0, psll1, onesrow,
             h2_ref, o0, o1, o2, o3):
    cntc = jnp.maximum(cnt[...], 1.0)
    mu = mus[...] / cntc
    var = vars_[...] / cntc
    oh = oh_ref[...]
    sub = hn_ref[...] - ms[...] * (oh @ mu)
    hnorm = gw[...] * sub / jnp.sqrt(oh @ var + 1e-5) + gb[...]
    h2 = jnp.where(hnorm > 0, hnorm, jnp.exp(jnp.minimum(hnorm, 0.0)) - 1.0)
    h2_ref[...] = h2
    sel = (bmat[...], pxl0[...], pxl1[...], psll0[...], psll1[...],
           onesrow[...])
    t = _tables_from_h(h2, Wl[...], Wr[...], We[...], att[...], eam[0, 0],
                       sel)
    o0[...], o1[...], o2[...], o3[...] = t


def _e1l2_kern(o0, o1, h_ref, oh_ref, Wres, bias, pn0, pn1, db0, db1,
               gm_ref):
    h3 = _assemble_h(o0[...], o1[...], pn0[...], pn1[...], db0[...],
                     db1[...]) + h_ref[...] @ Wres[...] + bias[...]

    @pl.when(pl.program_id(0) == 0)
    def _():
        gm_ref[...] = jnp.zeros_like(gm_ref)

    gm_ref[...] += lax.dot_general(oh_ref[...], h3, (((0,), (0,)), ((), ())))


def _head_kern(gms, cnt, demo, dW, db, c1w, c1b, b1w, b1b,
               c2w, c2b, b2w, b2b, c3w, c3b, o_ref):
    gm = gms[...] / jnp.maximum(cnt[...], 1.0)
    dm = demo[...] @ dW[...] + db[...]
    r64 = lax.broadcasted_iota(jnp.int32, (64, 80), 0)
    c64 = lax.broadcasted_iota(jnp.int32, (64, 80), 1)
    e1 = (r64 == c64).astype(_f32)
    r16 = lax.broadcasted_iota(jnp.int32, (16, 80), 0)
    c16 = lax.broadcasted_iota(jnp.int32, (16, 80), 1)
    e2 = (r16 + 64 == c16).astype(_f32)
    z = gm @ e1 + dm @ e2
    isq = 1.0 / jnp.sqrt(1.0 + 1e-5)
    z = z @ c1w[...] + c1b[...]
    z = z * (isq * b1w[...]) + b1b[...]
    z = jnp.where(z >= 0, z, 0.01 * z)
    z = z @ c2w[...] + c2b[...]
    z = z * (isq * b2w[...]) + b2b[...]
    z = jnp.where(z >= 0, z, 0.01 * z)
    o_ref[...] = z @ c3w[...] + c3b[...]


def _row_spec(w):
    return pl.BlockSpec((BN, w), lambda i: (i, 0))


def _full_spec(shape):
    return pl.BlockSpec(shape, lambda i: (0, 0))


def _acc_spec():
    return pl.BlockSpec((G, HC), lambda i: (0, 0))


_SDS = jax.ShapeDtypeStruct


def _sel_specs():
    return [_full_spec((HC, HC)), _full_spec((HC, ROW)), _full_spec((HC, ROW)),
            _full_spec((HC, ROW)), _full_spec((HC, ROW)), _full_spec((1, ROW))]


def _pn_specs():
    return [_full_spec((ROW, HC)), _full_spec((ROW, HC)),
            _full_spec((ROW, HC)), _full_spec((ROW, HC))]


def _k1(h, Wl, Wr, We, att_flat, eam):
    din = h.shape[1]
    return pl.pallas_call(
        _k1_kern,
        grid=(NBLK,),
        in_specs=[_row_spec(din), _full_spec((din, HC)), _full_spec((din, HC)),
                  _full_spec((1, HC)), _full_spec((1, HC)), _full_spec((1, 1))]
                 + _sel_specs(),
        out_specs=[_row_spec(ROW)] * 4,
        out_shape=[_SDS((N, ROW), _f32)] * 4,
    )(h, Wl, Wr, We, att_flat, eam, *_sel_args())


def _e1(o0, o1, h, oh, Wres, bias):
    din = h.shape[1]
    return pl.pallas_call(
        _e1_kern,
        grid=(NBLK,),
        in_specs=[_row_spec(ROW), _row_spec(ROW), _row_spec(din),
                  _row_spec(G), _full_spec((din, HC)), _full_spec((1, HC))]
                 + _pn_specs(),
        out_specs=[_row_spec(HC), _acc_spec(), _acc_spec()],
        out_shape=[_SDS((N, HC), _f32), _SDS((G, HC), _f32),
                   _SDS((G, HC), _f32)],
    )(o0, o1, h, oh, Wres, bias, *_pn_args())


def _e2(hn, oh, mus, cnt, ms):
    return pl.pallas_call(
        _e2_kern,
        grid=(NBLK,),
        in_specs=[_row_spec(HC), _row_spec(G), _full_spec((G, HC)),
                  _full_spec((G, HC)), _full_spec((1, HC))],
        out_specs=_acc_spec(),
        out_shape=_SDS((G, HC), _f32),
    )(hn, oh, mus, cnt, ms)


def _e3(hn, oh, mus, cnt, var, gw, gb, ms, Wl, Wr, We, att_flat, eam):
    return pl.pallas_call(
        _e3_kern,
        grid=(NBLK,),
        in_specs=[_row_spec(HC), _row_spec(G), _full_spec((G, HC)),
                  _full_spec((G, HC)), _full_spec((G, HC)),
                  _full_spec((1, HC)), _full_spec((1, HC)), _full_spec((1, HC)),
                  _full_spec((HC, HC)), _full_spec((HC, HC)),
                  _full_spec((1, HC)), _full_spec((1, HC)), _full_spec((1, 1))]
                 + _sel_specs(),
        out_specs=[_row_spec(HC)] + [_row_spec(ROW)] * 4,
        out_shape=[_SDS((N, HC), _f32)] + [_SDS((N, ROW), _f32)] * 4,
    )(hn, oh, mus, cnt, var, gw, gb, ms, Wl, Wr, We, att_flat, eam,
      *_sel_args())


def _e1l2(o0, o1, h, oh, Wres, bias):
    return pl.pallas_call(
        _e1l2_kern,
        grid=(NBLK,),
        in_specs=[_row_spec(ROW), _row_spec(ROW), _row_spec(HC),
                  _row_spec(G), _full_spec((HC, HC)), _full_spec((1, HC))]
                 + _pn_specs(),
        out_specs=_acc_spec(),
        out_shape=_SDS((G, HC), _f32),
    )(o0, o1, h, oh, Wres, bias, *_pn_args())


def kernel(x, edge_index, edge_attr, batch, demographics, emb,
           l0_Wl, l0_Wr, l0_We, l0_att, l0_b, l0_Wres,
           l1_Wl, l1_Wr, l1_We, l1_att, l1_b, l1_Wres,
           l2_Wl, l2_Wr, l2_We, l2_att, l2_b, l2_Wres,
           gn0_w, gn0_b, gn0_ms, gn1_w, gn1_b, gn1_ms,
           demo_W, demo_b, c1_W, c1_b, bn1_w, bn1_b,
           c2_W, c2_b, bn2_w, bn2_b, c3_W, c3_b):
    src = edge_index[0].astype(jnp.int32)
    dst = edge_index[1].astype(jnp.int32)
    ea = edge_attr.reshape(E)
    xpad = jnp.concatenate([x.astype(jnp.int32),
                            jnp.zeros((NP - N,), jnp.int32)])
    onehot = (batch[:, None] == jnp.arange(G)[None, :]).astype(_f32)

    easum = pl.pallas_call(
        _sum_kern,
        grid=(1,),
        in_specs=[pl.BlockSpec((6250, 128), lambda i: (0, 0))],
        out_specs=pl.BlockSpec((8, 128), lambda i: (0, 0)),
        out_shape=_SDS((8, 128), _f32),
    )(ea.reshape(6250, 128))
    eam = easum[0:1, 0:1] / float(E)

    h = _emb_gather(emb, xpad)[:N]

    layers = [
        (l0_Wl, l0_Wr, l0_We, l0_att, l0_b, l0_Wres),
        (l1_Wl, l1_Wr, l1_We, l1_att, l1_b, l1_Wres),
        (l2_Wl, l2_Wr, l2_We, l2_att, l2_b, l2_Wres),
    ]
    gns = [(gn0_w, gn0_b, gn0_ms), (gn1_w, gn1_b, gn1_ms)]

    def _const_row(We, att):
        return jnp.stack([
            jnp.concatenate([We[0, :32], att[:2].reshape(32)]),
            jnp.concatenate([We[0, 32:], att[2:].reshape(32)]),
        ])

    cnt = None
    musum = None
    tabs = _k1(h, l0_Wl, l0_Wr, l0_We, l0_att.reshape(1, HC), eam)
    for i in range(3):
        Wl, Wr, We, att, b, Wres = layers[i]
        constT = _const_row(We, att)
        o0, o1 = _edge_call(tabs[0], tabs[1], tabs[2], tabs[3],
                            src, dst, ea, constT)
        if i < 2:
            hn, musum, cnt_i = _e1(o0, o1, h, onehot, Wres, b.reshape(1, HC))
            if cnt is None:
                cnt = cnt_i
            gw, gb, ms = gns[i]
            var = _e2(hn, onehot, musum, cnt, ms.reshape(1, HC))
            nWl, nWr, nWe, natt = layers[i + 1][:4]
            res = _e3(hn, onehot, musum, cnt, var,
                      gw.reshape(1, HC), gb.reshape(1, HC), ms.reshape(1, HC),
                      nWl, nWr, nWe, natt.reshape(1, HC), eam)
            h = res[0]
            tabs = res[1:]
        else:
            gmsum = _e1l2(o0, o1, h, onehot, Wres, b.reshape(1, HC))

    out = pl.pallas_call(
        _head_kern,
        grid=(1,),
        in_specs=[_full_spec((G, HC)), _full_spec((G, HC)),
                  _full_spec((G, 4)), _full_spec((4, 16)), _full_spec((1, 16)),
                  _full_spec((80, 64)), _full_spec((1, 64)),
                  _full_spec((1, 64)), _full_spec((1, 64)),
                  _full_spec((64, 32)), _full_spec((1, 32)),
                  _full_spec((1, 32)), _full_spec((1, 32)),
                  _full_spec((32, 2)), _full_spec((1, 2))],
        out_specs=_full_spec((G, 2)),
        out_shape=_SDS((G, 2), _f32),
    )(gmsum, cnt, demographics, demo_W, demo_b.reshape(1, 16),
      c1_W, c1_b.reshape(1, 64), bn1_w.reshape(1, 64), bn1_b.reshape(1, 64),
      c2_W, c2_b.reshape(1, 32), bn2_w.reshape(1, 32), bn2_b.reshape(1, 32),
      c3_W, c3_b.reshape(1, 2))
    return out


# speedup vs baseline: 32.5232x; 1.1998x over previous
import numpy as np
import jax
import jax.numpy as jnp
from jax import lax
from jax.experimental import pallas as pl
from jax.experimental.pallas import tpu as pltpu
from jax.experimental.pallas import tpu_sc as plsc

N = 50000
E = 800000
G = 16
H = 4
C = 16
HC = 64

NC = 2
NS = 16
L = 16
ROW = 40
BE = 16
NBAT = E // BE
NBT = NBAT // NS
RPT = N // NS
NP = 50048
BN = 1000
NBLK = N // BN

_f32 = jnp.float32


def _col(k):
    return k + 4 * (k // 16)


def _selectors():
    Pg = []
    Pxl = []
    Psll = []
    Pn = []
    Db = []
    for c in range(2):
        pxl = np.zeros((64, ROW), np.float32)
        psll = np.zeros((64, ROW), np.float32)
        pn = np.zeros((ROW, 64), np.float32)
        db = np.zeros((ROW, 64), np.float32)
        pg = np.zeros((64, 32), np.float32)
        for k in range(32):
            pg[32 * c + k, k] = 1.0
            pxl[32 * c + k, _col(k)] = 1.0
            pn[_col(k), 32 * c + k] = 1.0
            db[16 + 20 * (k // 16), 32 * c + k] = 1.0
        psll[32 * c, 16] = 1.0
        psll[32 * c + 16, 36] = 1.0
        Pg.append(pg)
        Pxl.append(pxl)
        Psll.append(psll)
        Pn.append(pn)
        Db.append(db)
    ones_row = np.zeros((1, ROW), np.float32)
    ones_row[0, 16] = 1.0
    ones_row[0, 36] = 1.0
    Bmat = np.kron(np.eye(4, dtype=np.float32), np.ones((16, 16), np.float32))
    return Pg, Pxl, Psll, Pn, Db, ones_row, Bmat


_PG, _PXL, _PSLL, _PN, _DB, _ONESROW, _BMAT = _selectors()


def _sel_args():
    return (jnp.asarray(_BMAT), jnp.asarray(_PXL[0]), jnp.asarray(_PXL[1]),
            jnp.asarray(_PSLL[0]), jnp.asarray(_PSLL[1]),
            jnp.asarray(_ONESROW), jnp.asarray(_PG[0]), jnp.asarray(_PG[1]))


def _pn_args():
    return (jnp.asarray(_PN[0]), jnp.asarray(_PN[1]),
            jnp.asarray(_DB[0]), jnp.asarray(_DB[1]))


_SEL_SPECS = [None, None, None, None, None, None]
_PN_SPECS = [None, None, None, None]


def _emb_gather_body(emb_h, x_h, out_h, idx_v, rows_v, sem):
    c = lax.axis_index("c")
    s = lax.axis_index("s")
    wid = s * NC + c

    def body(j, carry):
        bid = wid + NC * NS * j

        @pl.when(bid < NP // BE)
        def _():
            off = bid * BE
            pltpu.sync_copy(x_h.at[pl.ds(off, BE)], idx_v)
            pltpu.async_copy(emb_h.at[idx_v], rows_v, sem).wait()
            pltpu.sync_copy(rows_v, out_h.at[pl.ds(off, BE)])

        return carry

    lax.fori_loop(0, -(-(NP // BE) // (NC * NS)), body, 0)


def _emb_gather(emb, xpad):
    mesh = plsc.VectorSubcoreMesh(core_axis_name="c", subcore_axis_name="s")
    return pl.kernel(
        _emb_gather_body,
        out_type=jax.ShapeDtypeStruct((NP, 16), _f32),
        mesh=mesh,
        compiler_params=pltpu.CompilerParams(use_tc_tiling_on_sc=False, needs_layout_passes=False),
        scratch_types=[
            pltpu.VMEM((BE,), jnp.int32),
            pltpu.VMEM((BE, 16), _f32),
            pltpu.SemaphoreType.DMA,
        ],
    )(emb, xpad)


EB = 48 * 4
XLB = BE * 32 * 4
XRB = BE * ROW * 4
SCB = BE * ROW * 4


def _edge_body(xlg0, xlg1, xlT0, xlT1, xr0, xr1, edata, constr,
               out0, out1,
               acc, xl_st, xr_st, out_st, ebuf, cvm,
               es0, es1, ga0, ga1, gb0, gb1, ss0, ss1):
    c = lax.axis_index("c")
    s = lax.axis_index("s")
    pltpu.sync_copy(constr.at[c], cvm)

    def _blkcopy(src_ref, dst_ref):
        pltpu.sync_copy(src_ref.at[pl.ds(s * 2960, 2960)],
                        dst_ref.at[pl.ds(s * 2960, 2960)])

        @pl.when(s == 0)
        def _():
            pltpu.sync_copy(src_ref.at[pl.ds(47360, 2640)],
                            dst_ref.at[pl.ds(47360, 2640)])

    @pl.when(c == 0)
    def _():
        _blkcopy(xlT0, acc)

    @pl.when(c == 1)
    def _():
        _blkcopy(xlT1, acc)

    riota = lax.iota(jnp.int32, L)
    zv = jnp.zeros((L,), _f32)
    for p in range(2):
        for col in (17, 18, 19, 37, 38, 39):
            colv = jnp.full((L,), col, jnp.int32)
            plsc.store_scatter(out_st.at[p], [riota, colv], zv)

    plsc.subcore_barrier()

    wevs = [cvm[pl.ds(0, 16)], cvm[pl.ds(16, 16)]]
    atvs = [cvm[pl.ds(32, 16)], cvm[pl.ds(48, 16)]]
    esems = [es0, es1]
    gasems = [ga0, ga1]
    gbsems = [gb0, gb1]
    ssems = [ss0, ss1]
    base = s * NBT

    def eload(b, p):
        pltpu.async_copy(edata.at[base + b], ebuf.at[p], esems[p])

    def gissue(p):
        srcv = ebuf[p, pl.ds(0, L)]
        dstv = ebuf[p, pl.ds(L, L)]

        @pl.when(c == 0)
        def _():
            pltpu.async_copy(xlg0.at[srcv], xl_st.at[p], gasems[p])
            pltpu.async_copy(xr0.at[dstv], xr_st.at[p], gbsems[p])

        @pl.when(c == 1)
        def _():
            pltpu.async_copy(xlg1.at[srcv], xl_st.at[p], gasems[p])
            pltpu.async_copy(xr1.at[dstv], xr_st.at[p], gbsems[p])

    def compute_scatter(p):
        dstv = ebuf[p, pl.ds(L, L)]
        eav = plsc.bitcast(ebuf[p, pl.ds(2 * L, L)], _f32)
        pltpu.make_async_copy(xlg0.at[pl.ds(0, BE)], xl_st.at[p],
                              gasems[p]).wait()
        pltpu.make_async_copy(xr0.at[pl.ds(0, BE)], xr_st.at[p],
                              gbsems[p]).wait()


        for hh in range(2):
            basec = 20 * hh
            bxl = 16 * hh
            logit = jnp.zeros((L,), _f32)
            xls = []
            for ch in range(16):
                xlc = plsc.load_gather(xl_st.at[p],
                                       [riota, jnp.full((L,), bxl + ch,
                                                        jnp.int32)])
                xrc = plsc.load_gather(xr_st.at[p],
                                       [riota, jnp.full((L,), basec + ch,
                                                        jnp.int32)])
                pre = xlc + xrc + eav * wevs[hh][ch]
                m = jnp.maximum(pre, 0.2 * pre)
                logit = logit + m * atvs[hh][ch]
                xls.append(xlc)
            dencol = jnp.full((L,), basec + 16, jnp.int32)
            sll = plsc.load_gather(xr_st.at[p], [riota, dencol])
            ex = jnp.exp(logit - sll)
            for ch in range(16):
                colv = jnp.full((L,), basec + ch, jnp.int32)
                plsc.store_scatter(out_st.at[p], [riota, colv], ex * xls[ch])
            plsc.store_scatter(out_st.at[p], [riota, dencol], ex)
        pltpu.sync_copy(out_st.at[p], acc.at[dstv], add=True)

    eload(0, 0)
    eload(1, 1)
    pltpu.make_async_copy(edata.at[0], ebuf.at[0], esems[0]).wait()
    gissue(0)

    def pair(jj, carry):
        for p in (0, 1):
            b = 2 * jj + p

            @pl.when(b + 1 < NBT)
            def _():
                pltpu.make_async_copy(edata.at[0], ebuf.at[1 - p],
                                      esems[1 - p]).wait()
                gissue(1 - p)

            compute_scatter(p)

            @pl.when(b + 2 < NBT)
            def _():
                eload(b + 2, p)
        return carry

    lax.fori_loop(0, NBT // 2, pair, 0)
    compute_scatter(0)
    plsc.subcore_barrier()

    @pl.when(c == 0)
    def _():
        _blkcopy(acc, out0)

    @pl.when(c == 1)
    def _():
        _blkcopy(acc, out1)


def _edge_call(xlg0, xlg1, xlT0, xlT1, xr0, xr1, edata, constT):
    mesh = plsc.VectorSubcoreMesh(core_axis_name="c", subcore_axis_name="s")
    return pl.kernel(
        _edge_body,
        out_type=[
            jax.ShapeDtypeStruct((N, ROW), _f32),
            jax.ShapeDtypeStruct((N, ROW), _f32),
        ],
        mesh=mesh,
        compiler_params=pltpu.CompilerParams(use_tc_tiling_on_sc=False, needs_layout_passes=False),
        scratch_types=[
            pltpu.VMEM_SHARED((N, ROW), _f32),
            pltpu.VMEM((2, BE, 32), _f32),
            pltpu.VMEM((2, BE, ROW), _f32),
            pltpu.VMEM((2, BE, ROW), _f32),
            pltpu.VMEM((2, 48), jnp.int32),
            pltpu.VMEM((64,), _f32),
            pltpu.SemaphoreType.DMA,
            pltpu.SemaphoreType.DMA,
            pltpu.SemaphoreType.DMA,
            pltpu.SemaphoreType.DMA,
            pltpu.SemaphoreType.DMA,
            pltpu.SemaphoreType.DMA,
            pltpu.SemaphoreType.DMA,
            pltpu.SemaphoreType.DMA,
        ],
    )(xlg0, xlg1, xlT0, xlT1, xr0, xr1, edata, constT)


def _sum_kern(ea_ref, o_ref):
    @pl.when(pl.program_id(0) == 0)
    def _():
        o_ref[...] = jnp.zeros_like(o_ref)

    o_ref[...] += jnp.sum(ea_ref[...])


def _tables_from_h(hb, Wl, Wr, We, att_flat, ea_mean, sel):
    bmat, pxl0, pxl1, psll0, psll1, onesrow, pg0, pg1 = sel
    xl = hb @ Wl
    xr = hb @ Wr
    pre = xl + xr + ea_mean * We
    m = jnp.maximum(pre, 0.2 * pre)
    sll_wide = (m * att_flat) @ bmat
    xlg0 = xl @ pg0
    xlg1 = xl @ pg1
    xlT0 = xl @ pxl0 + onesrow
    xlT1 = xl @ pxl1 + onesrow
    xrT0 = xr @ pxl0 + sll_wide @ psll0
    xrT1 = xr @ pxl1 + sll_wide @ psll1
    return xlg0, xlg1, xlT0, xlT1, xrT0, xrT1


def _k1_kern(h_ref, Wl, Wr, We, att, eam,
             bmat, pxl0, pxl1, psll0, psll1, onesrow, pg0, pg1,
             og0, og1, o0, o1, o2, o3):
    sel = (bmat[...], pxl0[...], pxl1[...], psll0[...], psll1[...],
           onesrow[...], pg0[...], pg1[...])
    t = _tables_from_h(h_ref[...], Wl[...], Wr[...], We[...], att[...],
                       eam[0, 0], sel)
    og0[...], og1[...], o0[...], o1[...], o2[...], o3[...] = t


def _assemble_h(o0b, o1b, pn0, pn1, db0, db1):
    h = jnp.zeros((o0b.shape[0], HC), _f32)
    for ob, pn, db in ((o0b, pn0, db0), (o1b, pn1, db1)):
        h = h + (ob @ pn) / (ob @ db + 1e-16)
    return h


def _e1_kern(o0, o1, h_ref, oh_ref, Wres, bias, pn0, pn1, db0, db1,
             hn_ref, mu_ref, cnt_ref):
    hn = _assemble_h(o0[...], o1[...], pn0[...], pn1[...], db0[...],
                     db1[...]) + h_ref[...] @ Wres[...] + bias[...]
    hn_ref[...] = hn
    oh = oh_ref[...]

    @pl.when(pl.program_id(0) == 0)
    def _():
        mu_ref[...] = jnp.zeros_like(mu_ref)
        cnt_ref[...] = jnp.zeros_like(cnt_ref)

    mu_ref[...] += lax.dot_general(oh, hn, (((0,), (0,)), ((), ())))
    cnt_ref[...] += lax.dot_general(
        oh, jnp.ones_like(hn), (((0,), (0,)), ((), ())))


def _e2_kern(hn_ref, oh_ref, mus, cnt, ms, var_ref):
    cntc = jnp.maximum(cnt[...], 1.0)
    mu = mus[...] / cntc
    oh = oh_ref[...]
    sub = hn_ref[...] - ms[...] * (oh @ mu)

    @pl.when(pl.program_id(0) == 0)
    def _():
        var_ref[...] = jnp.zeros_like(var_ref)

    var_ref[...] += lax.dot_general(oh, sub * sub, (((0,), (0,)), ((), ())))


def _e3_kern(hn_ref, oh_ref, mus, cnt, vars_, gw, gb, ms,
             Wl, Wr, We, att, eam,
             bmat, pxl0, pxl1, psll0, psll1, onesrow, pg0, pg1,
             h2_ref, og0, og1, o0, o1, o2, o3):
    cntc = jnp.maximum(cnt[...], 1.0)
    mu = mus[...] / cntc
    var = vars_[...] / cntc
    oh = oh_ref[...]
    sub = hn_ref[...] - ms[...] * (oh @ mu)
    hnorm = gw[...] * sub / jnp.sqrt(oh @ var + 1e-5) + gb[...]
    h2 = jnp.where(hnorm > 0, hnorm, jnp.exp(jnp.minimum(hnorm, 0.0)) - 1.0)
    h2_ref[...] = h2
    sel = (bmat[...], pxl0[...], pxl1[...], psll0[...], psll1[...],
           onesrow[...], pg0[...], pg1[...])
    t = _tables_from_h(h2, Wl[...], Wr[...], We[...], att[...], eam[0, 0],
                       sel)
    og0[...], og1[...], o0[...], o1[...], o2[...], o3[...] = t


def _e1l2_kern(o0, o1, h_ref, oh_ref, Wres, bias, pn0, pn1, db0, db1,
               gm_ref):
    h3 = _assemble_h(o0[...], o1[...], pn0[...], pn1[...], db0[...],
                     db1[...]) + h_ref[...] @ Wres[...] + bias[...]

    @pl.when(pl.program_id(0) == 0)
    def _():
        gm_ref[...] = jnp.zeros_like(gm_ref)

    gm_ref[...] += lax.dot_general(oh_ref[...], h3, (((0,), (0,)), ((), ())))


def _head_kern(gms, cnt, demo, dW, db, c1w, c1b, b1w, b1b,
               c2w, c2b, b2w, b2b, c3w, c3b, o_ref):
    gm = gms[...] / jnp.maximum(cnt[...], 1.0)
    dm = demo[...] @ dW[...] + db[...]
    r64 = lax.broadcasted_iota(jnp.int32, (64, 80), 0)
    c64 = lax.broadcasted_iota(jnp.int32, (64, 80), 1)
    e1 = (r64 == c64).astype(_f32)
    r16 = lax.broadcasted_iota(jnp.int32, (16, 80), 0)
    c16 = lax.broadcasted_iota(jnp.int32, (16, 80), 1)
    e2 = (r16 + 64 == c16).astype(_f32)
    z = gm @ e1 + dm @ e2
    isq = 1.0 / jnp.sqrt(1.0 + 1e-5)
    z = z @ c1w[...] + c1b[...]
    z = z * (isq * b1w[...]) + b1b[...]
    z = jnp.where(z >= 0, z, 0.01 * z)
    z = z @ c2w[...] + c2b[...]
    z = z * (isq * b2w[...]) + b2b[...]
    z = jnp.where(z >= 0, z, 0.01 * z)
    o_ref[...] = z @ c3w[...] + c3b[...]


def _row_spec(w):
    return pl.BlockSpec((BN, w), lambda i: (i, 0))


def _full_spec(shape):
    return pl.BlockSpec(shape, lambda i: (0, 0))


def _acc_spec():
    return pl.BlockSpec((G, HC), lambda i: (0, 0))


_SDS = jax.ShapeDtypeStruct


def _sel_specs():
    return [_full_spec((HC, HC)), _full_spec((HC, ROW)), _full_spec((HC, ROW)),
            _full_spec((HC, ROW)), _full_spec((HC, ROW)), _full_spec((1, ROW)),
            _full_spec((HC, 32)), _full_spec((HC, 32))]


def _pn_specs():
    return [_full_spec((ROW, HC)), _full_spec((ROW, HC)),
            _full_spec((ROW, HC)), _full_spec((ROW, HC))]


def _k1(h, Wl, Wr, We, att_flat, eam):
    din = h.shape[1]
    return pl.pallas_call(
        _k1_kern,
        grid=(NBLK,),
        in_specs=[_row_spec(din), _full_spec((din, HC)), _full_spec((din, HC)),
                  _full_spec((1, HC)), _full_spec((1, HC)), _full_spec((1, 1))]
                 + _sel_specs(),
        out_specs=[_row_spec(32)] * 2 + [_row_spec(ROW)] * 4,
        out_shape=[_SDS((N, 32), _f32)] * 2 + [_SDS((N, ROW), _f32)] * 4,
    )(h, Wl, Wr, We, att_flat, eam, *_sel_args())


def _e1(o0, o1, h, oh, Wres, bias):
    din = h.shape[1]
    return pl.pallas_call(
        _e1_kern,
        grid=(NBLK,),
        in_specs=[_row_spec(ROW), _row_spec(ROW), _row_spec(din),
                  _row_spec(G), _full_spec((din, HC)), _full_spec((1, HC))]
                 + _pn_specs(),
        out_specs=[_row_spec(HC), _acc_spec(), _acc_spec()],
        out_shape=[_SDS((N, HC), _f32), _SDS((G, HC), _f32),
                   _SDS((G, HC), _f32)],
    )(o0, o1, h, oh, Wres, bias, *_pn_args())


def _e2(hn, oh, mus, cnt, ms):
    return pl.pallas_call(
        _e2_kern,
        grid=(NBLK,),
        in_specs=[_row_spec(HC), _row_spec(G), _full_spec((G, HC)),
                  _full_spec((G, HC)), _full_spec((1, HC))],
        out_specs=_acc_spec(),
        out_shape=_SDS((G, HC), _f32),
    )(hn, oh, mus, cnt, ms)


def _e3(hn, oh, mus, cnt, var, gw, gb, ms, Wl, Wr, We, att_flat, eam):
    return pl.pallas_call(
        _e3_kern,
        grid=(NBLK,),
        in_specs=[_row_spec(HC), _row_spec(G), _full_spec((G, HC)),
                  _full_spec((G, HC)), _full_spec((G, HC)),
                  _full_spec((1, HC)), _full_spec((1, HC)), _full_spec((1, HC)),
                  _full_spec((HC, HC)), _full_spec((HC, HC)),
                  _full_spec((1, HC)), _full_spec((1, HC)), _full_spec((1, 1))]
                 + _sel_specs(),
        out_specs=[_row_spec(HC)] + [_row_spec(32)] * 2 + [_row_spec(ROW)] * 4,
        out_shape=([_SDS((N, HC), _f32)] + [_SDS((N, 32), _f32)] * 2
                   + [_SDS((N, ROW), _f32)] * 4),
    )(hn, oh, mus, cnt, var, gw, gb, ms, Wl, Wr, We, att_flat, eam,
      *_sel_args())


def _e1l2(o0, o1, h, oh, Wres, bias):
    return pl.pallas_call(
        _e1l2_kern,
        grid=(NBLK,),
        in_specs=[_row_spec(ROW), _row_spec(ROW), _row_spec(HC),
                  _row_spec(G), _full_spec((HC, HC)), _full_spec((1, HC))]
                 + _pn_specs(),
        out_specs=_acc_spec(),
        out_shape=_SDS((G, HC), _f32),
    )(o0, o1, h, oh, Wres, bias, *_pn_args())


def kernel(x, edge_index, edge_attr, batch, demographics, emb,
           l0_Wl, l0_Wr, l0_We, l0_att, l0_b, l0_Wres,
           l1_Wl, l1_Wr, l1_We, l1_att, l1_b, l1_Wres,
           l2_Wl, l2_Wr, l2_We, l2_att, l2_b, l2_Wres,
           gn0_w, gn0_b, gn0_ms, gn1_w, gn1_b, gn1_ms,
           demo_W, demo_b, c1_W, c1_b, bn1_w, bn1_b,
           c2_W, c2_b, bn2_w, bn2_b, c3_W, c3_b):
    src = edge_index[0].astype(jnp.int32)
    dst = edge_index[1].astype(jnp.int32)
    ea = edge_attr.reshape(E)
    edata = jnp.concatenate(
        [src.reshape(NBAT, BE), dst.reshape(NBAT, BE),
         lax.bitcast_convert_type(ea, jnp.int32).reshape(NBAT, BE)], axis=1)
    xpad = jnp.concatenate([x.astype(jnp.int32),
                            jnp.zeros((NP - N,), jnp.int32)])
    onehot = (batch[:, None] == jnp.arange(G)[None, :]).astype(_f32)

    easum = pl.pallas_call(
        _sum_kern,
        grid=(1,),
        in_specs=[pl.BlockSpec((6250, 128), lambda i: (0, 0))],
        out_specs=pl.BlockSpec((8, 128), lambda i: (0, 0)),
        out_shape=_SDS((8, 128), _f32),
    )(ea.reshape(6250, 128))
    eam = easum[0:1, 0:1] / float(E)

    h = _emb_gather(emb, xpad)[:N]

    layers = [
        (l0_Wl, l0_Wr, l0_We, l0_att, l0_b, l0_Wres),
        (l1_Wl, l1_Wr, l1_We, l1_att, l1_b, l1_Wres),
        (l2_Wl, l2_Wr, l2_We, l2_att, l2_b, l2_Wres),
    ]
    gns = [(gn0_w, gn0_b, gn0_ms), (gn1_w, gn1_b, gn1_ms)]

    def _const_row(We, att):
        return jnp.stack([
            jnp.concatenate([We[0, :32], att[:2].reshape(32)]),
            jnp.concatenate([We[0, 32:], att[2:].reshape(32)]),
        ])

    cnt = None
    musum = None
    tabs = _k1(h, l0_Wl, l0_Wr, l0_We, l0_att.reshape(1, HC), eam)
    for i in range(3):
        Wl, Wr, We, att, b, Wres = layers[i]
        constT = _const_row(We, att)
        o0, o1 = _edge_call(tabs[0], tabs[1], tabs[2], tabs[3],
                            tabs[4], tabs[5], edata, constT)
        if i < 2:
            hn, musum, cnt_i = _e1(o0, o1, h, onehot, Wres, b.reshape(1, HC))
            if cnt is None:
                cnt = cnt_i
            gw, gb, ms = gns[i]
            var = _e2(hn, onehot, musum, cnt, ms.reshape(1, HC))
            nWl, nWr, nWe, natt = layers[i + 1][:4]
            res = _e3(hn, onehot, musum, cnt, var,
                      gw.reshape(1, HC), gb.reshape(1, HC), ms.reshape(1, HC),
                      nWl, nWr, nWe, natt.reshape(1, HC), eam)
            h = res[0]
            tabs = res[1:]
        else:
            gmsum = _e1l2(o0, o1, h, onehot, Wres, b.reshape(1, HC))

    out = pl.pallas_call(
        _head_kern,
        grid=(1,),
        in_specs=[_full_spec((G, HC)), _full_spec((G, HC)),
                  _full_spec((G, 4)), _full_spec((4, 16)), _full_spec((1, 16)),
                  _full_spec((80, 64)), _full_spec((1, 64)),
                  _full_spec((1, 64)), _full_spec((1, 64)),
                  _full_spec((64, 32)), _full_spec((1, 32)),
                  _full_spec((1, 32)), _full_spec((1, 32)),
                  _full_spec((32, 2)), _full_spec((1, 2))],
        out_specs=_full_spec((G, 2)),
        out_shape=_SDS((G, 2), _f32),
    )(gmsum, cnt, demographics, demo_W, demo_b.reshape(1, 16),
      c1_W, c1_b.reshape(1, 64), bn1_w.reshape(1, 64), bn1_b.reshape(1, 64),
      c2_W, c2_b.reshape(1, 32), bn2_w.reshape(1, 32), bn2_b.reshape(1, 32),
      c3_W, c3_b.reshape(1, 2))
    return out
